# Initial kernel scaffold; baseline (speedup 1.0000x reference)
import jax
import jax.numpy as jnp
from jax.experimental import pallas as pl


def kernel(conv1_w, conv1_b, conv2_w, conv2_b, conv3_w, conv3_b, conv4_w, conv4_b, conv5_w, conv5_b, hidden_w, hidden_b, out_w, out_b, x):
    raise NotImplementedError("write your pallas kernel here")


# speedup vs baseline: 6.9917x; 6.9917x over previous
import functools

import jax
import jax.numpy as jnp
from jax.experimental import pallas as pl
from jax.experimental.pallas import tpu as pltpu


def _band(w, win, wout, pad):
    k = w.shape[0]
    cin, cout = w.shape[3], w.shape[4]
    d = jnp.arange(win)[:, None] - jnp.arange(wout)[None, :]
    mask = (d >= 0) & (d < k)
    g = w[:, :, jnp.clip(d, 0, k - 1)]
    g = g * mask[None, None, :, :, None, None]
    g = g.transpose(0, 1, 2, 4, 3, 5)
    g = g.reshape(k, k, win * cin, wout * cout)
    if pad > win * cin:
        g = jnp.pad(g, ((0, 0), (0, 0), (0, pad - win * cin), (0, 0)))
    return g.reshape(k * k * pad, wout * cout).astype(jnp.bfloat16)


def _btile(b, wout):
    return jnp.tile(b, wout).reshape(1, wout * b.shape[0]).astype(jnp.float32)


def _taps(y, k, dout, hout):
    return jnp.concatenate(
        [y[kd:kd + dout, kh:kh + hout, :] for kd in range(k) for kh in range(k)],
        axis=-1)


def _net_kernel(x_ref, t1, b1, t2, b2, t3, b3, t4, b4, t5, b5,
                wh, bh, wo, bo, o_ref):
    f32 = jnp.float32
    bf16 = jnp.bfloat16
    x = x_ref[...]

    xh = jnp.concatenate([x[:, kh:kh + 28, :] for kh in range(5)], axis=-1)
    a1 = jnp.concatenate([xh[kd:kd + 28] for kd in range(5)], axis=-1)
    y1 = jnp.dot(a1.reshape(784, 800), t1[...], preferred_element_type=f32)
    y1 = jnp.maximum(y1 + b1[...], 0.0).astype(bf16)
    y1 = jnp.concatenate([y1, jnp.zeros((784, 32), bf16)], axis=-1)
    y1 = y1.reshape(28, 28, 256)

    a2 = _taps(y1, 5, 24, 24)
    y2 = jnp.dot(a2.reshape(576, 6400), t2[...], preferred_element_type=f32)
    y2 = jnp.maximum(y2 + b2[...], 0.0).reshape(24, 24, 384)

    pe = jnp.concatenate([y2[:, :, 32 * w:32 * w + 16] for w in range(12)], -1)
    po = jnp.concatenate([y2[:, :, 32 * w + 16:32 * w + 32] for w in range(12)], -1)
    p = jnp.maximum(pe, po)
    p = p.reshape(12, 2, 24, 192)
    p = jnp.maximum(p[:, 0], p[:, 1]).reshape(12, 12, 2, 192)
    p = jnp.maximum(p[:, :, 0], p[:, :, 1])
    z = jnp.concatenate([p.astype(bf16), jnp.zeros((12, 12, 64), bf16)], -1)

    a3 = _taps(z, 4, 9, 9)
    y3 = jnp.dot(a3.reshape(81, 4096), t3[...], preferred_element_type=f32)
    y3 = jnp.maximum(y3 + b3[...], 0.0).astype(bf16)
    y3 = jnp.concatenate([y3, jnp.zeros((81, 96), bf16)], -1).reshape(9, 9, 384)

    a4 = _taps(y3, 4, 6, 6)
    y4 = jnp.dot(a4.reshape(36, 6144), t4[...], preferred_element_type=f32)
    y4 = jnp.maximum(y4 + b4[...], 0.0).reshape(6, 6, 384)

    pe = jnp.concatenate([y4[:, :, 128 * w:128 * w + 64] for w in range(3)], -1)
    po = jnp.concatenate([y4[:, :, 128 * w + 64:128 * w + 128] for w in range(3)], -1)
    p = jnp.maximum(pe, po)
    p = p.reshape(3, 2, 6, 192)
    p = jnp.maximum(p[:, 0], p[:, 1]).reshape(3, 3, 2, 192)
    p = jnp.maximum(p[:, :, 0], p[:, :, 1])
    z5 = jnp.concatenate([p.astype(bf16), jnp.zeros((3, 3, 64), bf16)], -1)

    a5 = _taps(z5, 2, 2, 2)
    y5 = jnp.dot(a5.reshape(4, 1024), t5[...], preferred_element_type=f32)
    y5 = jnp.maximum(y5 + b5[...], 0.0)

    feat = jnp.max(y5, axis=0, keepdims=True)
    feat = jnp.maximum(feat[:, 0:32], feat[:, 32:64])
    h = jnp.maximum(
        jnp.dot(feat, wh[...], preferred_element_type=f32) + bh[...], 0.0)
    o_ref[...] = jnp.dot(h, wo[...], preferred_element_type=f32) + bo[...]


def kernel(conv1_w, conv1_b, conv2_w, conv2_b, conv3_w, conv3_b,
           conv4_w, conv4_b, conv5_w, conv5_b, hidden_w, hidden_b,
           out_w, out_b, x):
    B = x.shape[0]
    xb = x.astype(jnp.bfloat16)

    t1 = _band(conv1_w, 32, 28, 32)
    t2 = _band(conv2_w, 28, 24, 256)
    t3 = _band(conv3_w, 12, 9, 256)
    t4 = _band(conv4_w, 9, 6, 384)
    t5 = _band(conv5_w, 3, 2, 256)
    b1 = _btile(conv1_b, 28)
    b2 = _btile(conv2_b, 24)
    b3 = _btile(conv3_b, 9)
    b4 = _btile(conv4_b, 6)
    b5 = _btile(conv5_b, 2)
    bh = hidden_b.reshape(1, -1).astype(jnp.float32)
    bo = out_b.reshape(1, -1).astype(jnp.float32)

    def cspec(arr):
        return pl.BlockSpec(arr.shape, lambda i: (0,) * arr.ndim)

    consts = [t1, b1, t2, b2, t3, b3, t4, b4, t5, b5,
              hidden_w, bh, out_w, bo]
    out = pl.pallas_call(
        _net_kernel,
        out_shape=jax.ShapeDtypeStruct((B, 1, 10), jnp.float32),
        grid=(B,),
        in_specs=[pl.BlockSpec((None, 32, 32, 32), lambda i: (i, 0, 0, 0))]
                 + [cspec(a) for a in consts],
        out_specs=pl.BlockSpec((None, 1, 10), lambda i: (i, 0, 0)),
        compiler_params=pltpu.CompilerParams(
            dimension_semantics=("parallel",),
            vmem_limit_bytes=64 * 1024 * 1024),
    )(xb, *consts)
    return out.reshape(B, 10)


# speedup vs baseline: 8.2630x; 1.1818x over previous
import functools

import jax
import jax.numpy as jnp
from jax.experimental import pallas as pl
from jax.experimental.pallas import tpu as pltpu


def _band(w, win, wout, pad, npad=0):
    k = w.shape[0]
    cin, cout = w.shape[3], w.shape[4]
    d = jnp.arange(win)[:, None] - jnp.arange(wout)[None, :]
    mask = (d >= 0) & (d < k)
    g = w[:, :, jnp.clip(d, 0, k - 1)]
    g = g * mask[None, None, :, :, None, None]
    g = g.transpose(0, 1, 2, 4, 3, 5)
    g = g.reshape(k, k, win * cin, wout * cout)
    if pad > win * cin:
        g = jnp.pad(g, ((0, 0), (0, 0), (0, pad - win * cin), (0, 0)))
    g = g.reshape(k * k * pad, wout * cout)
    if npad > wout * cout:
        g = jnp.pad(g, ((0, 0), (0, npad - wout * cout)))
    return g.astype(jnp.bfloat16)


def _btile(b, wout, npad=0):
    t = jnp.tile(b, wout)
    if npad > t.shape[0]:
        t = jnp.pad(t, (0, npad - t.shape[0]))
    return t.reshape(1, -1).astype(jnp.float32)


_BB = 2


def _taps(y, k, dout, hout):
    return jnp.concatenate(
        [y[:, kd:kd + dout, kh:kh + hout, :]
         for kd in range(k) for kh in range(k)],
        axis=-1)


def _net_kernel(x_ref, t1, b1, t2, b2, t3, b3, t4, b4, t5, b5,
                wh, bh, wo, bo, o_ref):
    f32 = jnp.float32
    bf16 = jnp.bfloat16
    bb = _BB
    x = x_ref[...]

    xh = jnp.concatenate([x[:, :, kh:kh + 28, :] for kh in range(5)], axis=-1)
    a1 = jnp.concatenate([xh[:, kd:kd + 28] for kd in range(5)], axis=-1)
    y1 = jnp.dot(a1.reshape(bb * 784, 800), t1[...], preferred_element_type=f32)
    y1 = jnp.maximum(y1 + b1[...], 0.0).astype(bf16)
    y1 = y1.reshape(bb, 28, 28, 256)

    a2 = _taps(y1, 5, 24, 24)
    y2 = jnp.dot(a2.reshape(bb * 576, 6400), t2[...], preferred_element_type=f32)
    y2 = jnp.maximum(y2 + b2[...], 0.0).astype(bf16).reshape(bb, 24, 24, 384)

    pe = jnp.concatenate([y2[..., 32 * w:32 * w + 16] for w in range(12)], -1)
    po = jnp.concatenate([y2[..., 32 * w + 16:32 * w + 32] for w in range(12)], -1)
    p = jnp.maximum(pe, po)
    p = p.reshape(bb, 12, 2, 24, 192)
    p = jnp.maximum(p[:, :, 0], p[:, :, 1]).reshape(bb, 12, 12, 2, 192)
    p = jnp.maximum(p[:, :, :, 0], p[:, :, :, 1])
    z = jnp.concatenate([p, jnp.zeros((bb, 12, 12, 64), bf16)], -1)

    a3 = _taps(z, 4, 9, 9)
    y3 = jnp.dot(a3.reshape(bb * 81, 4096), t3[...], preferred_element_type=f32)
    y3 = jnp.maximum(y3 + b3[...], 0.0).astype(bf16)
    y3 = y3.reshape(bb, 9, 9, 384)

    a4 = _taps(y3, 4, 6, 6)
    y4 = jnp.dot(a4.reshape(bb * 36, 6144), t4[...], preferred_element_type=f32)
    y4 = jnp.maximum(y4 + b4[...], 0.0).astype(bf16).reshape(bb, 6, 6, 384)

    pe = jnp.concatenate([y4[..., 128 * w:128 * w + 64] for w in range(3)], -1)
    po = jnp.concatenate([y4[..., 128 * w + 64:128 * w + 128] for w in range(3)], -1)
    p = jnp.maximum(pe, po)
    p = p.reshape(bb, 3, 2, 6, 192)
    p = jnp.maximum(p[:, :, 0], p[:, :, 1]).reshape(bb, 3, 3, 2, 192)
    p = jnp.maximum(p[:, :, :, 0], p[:, :, :, 1])
    z5 = jnp.concatenate([p, jnp.zeros((bb, 3, 3, 64), bf16)], -1)

    a5 = _taps(z5, 2, 2, 2)
    y5 = jnp.dot(a5.reshape(bb * 4, 1024), t5[...], preferred_element_type=f32)
    y5 = jnp.maximum(y5 + b5[...], 0.0)

    feat = jnp.max(y5.reshape(bb, 4, 64), axis=1)
    feat = jnp.maximum(feat[:, 0:32], feat[:, 32:64])
    h = jnp.maximum(
        jnp.dot(feat, wh[...], preferred_element_type=f32) + bh[...], 0.0)
    o_ref[...] = jnp.dot(h, wo[...], preferred_element_type=f32) + bo[...]


def kernel(conv1_w, conv1_b, conv2_w, conv2_b, conv3_w, conv3_b,
           conv4_w, conv4_b, conv5_w, conv5_b, hidden_w, hidden_b,
           out_w, out_b, x):
    B = x.shape[0]
    nb = B // _BB
    xb = x.astype(jnp.bfloat16).reshape(nb, _BB, 32, 32, 32)

    t1 = _band(conv1_w, 32, 28, 32, 256)
    t2 = _band(conv2_w, 28, 24, 256)
    t3 = _band(conv3_w, 12, 9, 256, 384)
    t4 = _band(conv4_w, 9, 6, 384)
    t5 = _band(conv5_w, 3, 2, 256)
    b1 = _btile(conv1_b, 28, 256)
    b2 = _btile(conv2_b, 24)
    b3 = _btile(conv3_b, 9, 384)
    b4 = _btile(conv4_b, 6)
    b5 = _btile(conv5_b, 2)
    bh = hidden_b.reshape(1, -1).astype(jnp.float32)
    bo = out_b.reshape(1, -1).astype(jnp.float32)

    def cspec(arr):
        return pl.BlockSpec(arr.shape, lambda i: (0,) * arr.ndim)

    consts = [t1, b1, t2, b2, t3, b3, t4, b4, t5, b5,
              hidden_w, bh, out_w, bo]
    out = pl.pallas_call(
        _net_kernel,
        out_shape=jax.ShapeDtypeStruct((nb, _BB, 10), jnp.float32),
        grid=(nb,),
        in_specs=[pl.BlockSpec((None, _BB, 32, 32, 32),
                               lambda i: (i, 0, 0, 0, 0))]
                 + [cspec(a) for a in consts],
        out_specs=pl.BlockSpec((None, _BB, 10), lambda i: (i, 0, 0)),
        compiler_params=pltpu.CompilerParams(
            dimension_semantics=("parallel",),
            vmem_limit_bytes=64 * 1024 * 1024),
    )(xb, *consts)
    return out.reshape(B, 10)


# speedup vs baseline: 8.5832x; 1.0387x over previous
import functools

import jax
import jax.numpy as jnp
from jax.experimental import pallas as pl
from jax.experimental.pallas import tpu as pltpu


def _band(w, win, wout, pad, npad=0):
    k = w.shape[0]
    cin, cout = w.shape[3], w.shape[4]
    d = jnp.arange(win)[:, None] - jnp.arange(wout)[None, :]
    mask = (d >= 0) & (d < k)
    g = w[:, :, jnp.clip(d, 0, k - 1)]
    g = g * mask[None, None, :, :, None, None]
    g = g.transpose(0, 1, 2, 4, 3, 5)
    g = g.reshape(k, k, win * cin, wout * cout)
    if pad > win * cin:
        g = jnp.pad(g, ((0, 0), (0, 0), (0, pad - win * cin), (0, 0)))
    g = g.reshape(k * k * pad, wout * cout)
    if npad > wout * cout:
        g = jnp.pad(g, ((0, 0), (0, npad - wout * cout)))
    return g.astype(jnp.bfloat16)


def _btile(b, wout, npad=0):
    t = jnp.tile(b, wout)
    if npad > t.shape[0]:
        t = jnp.pad(t, (0, npad - t.shape[0]))
    return t.reshape(1, -1).astype(jnp.float32)


_BB = 4


def _taps(y, k, dout, hout):
    return jnp.concatenate(
        [y[:, kd:kd + dout, kh:kh + hout, :]
         for kd in range(k) for kh in range(k)],
        axis=-1)


def _net_kernel(x_ref, t1, b1, t2, b2, t3, b3, t4, b4, t5, b5,
                wh, bh, wo, bo, o_ref):
    f32 = jnp.float32
    bf16 = jnp.bfloat16
    bb = _BB
    x = x_ref[...]

    xh = jnp.concatenate([x[:, :, kh:kh + 28, :] for kh in range(5)], axis=-1)
    a1 = jnp.concatenate([xh[:, kd:kd + 28] for kd in range(5)], axis=-1)
    y1 = jnp.dot(a1.reshape(bb * 784, 800), t1[...], preferred_element_type=f32)
    y1 = jnp.maximum(y1 + b1[...], 0.0).astype(bf16)
    y1 = y1.reshape(bb, 28, 28, 256)

    a2 = _taps(y1, 5, 24, 24)
    y2 = jnp.dot(a2.reshape(bb * 576, 6400), t2[...], preferred_element_type=f32)
    y2 = jnp.maximum(y2 + b2[...], 0.0).astype(bf16).reshape(bb, 24, 24, 384)

    pe = jnp.concatenate([y2[..., 32 * w:32 * w + 16] for w in range(12)], -1)
    po = jnp.concatenate([y2[..., 32 * w + 16:32 * w + 32] for w in range(12)], -1)
    p = jnp.maximum(pe, po)
    p = p.reshape(bb, 12, 2, 24, 192)
    p = jnp.maximum(p[:, :, 0], p[:, :, 1]).reshape(bb, 12, 12, 2, 192)
    p = jnp.maximum(p[:, :, :, 0], p[:, :, :, 1])
    z = jnp.concatenate([p, jnp.zeros((bb, 12, 12, 64), bf16)], -1)

    a3 = _taps(z, 4, 9, 9)
    y3 = jnp.dot(a3.reshape(bb * 81, 4096), t3[...], preferred_element_type=f32)
    y3 = jnp.maximum(y3 + b3[...], 0.0).astype(bf16)
    y3 = y3.reshape(bb, 9, 9, 384)

    a4 = _taps(y3, 4, 6, 6)
    y4 = jnp.dot(a4.reshape(bb * 36, 6144), t4[...], preferred_element_type=f32)
    y4 = jnp.maximum(y4 + b4[...], 0.0).astype(bf16).reshape(bb, 6, 6, 384)

    pe = jnp.concatenate([y4[..., 128 * w:128 * w + 64] for w in range(3)], -1)
    po = jnp.concatenate([y4[..., 128 * w + 64:128 * w + 128] for w in range(3)], -1)
    p = jnp.maximum(pe, po)
    p = p.reshape(bb, 3, 2, 6, 192)
    p = jnp.maximum(p[:, :, 0], p[:, :, 1]).reshape(bb, 3, 3, 2, 192)
    p = jnp.maximum(p[:, :, :, 0], p[:, :, :, 1])
    z5 = jnp.concatenate([p, jnp.zeros((bb, 3, 3, 64), bf16)], -1)

    a5 = _taps(z5, 2, 2, 2)
    y5 = jnp.dot(a5.reshape(bb * 4, 1024), t5[...], preferred_element_type=f32)
    y5 = jnp.maximum(y5 + b5[...], 0.0)

    feat = jnp.max(y5.reshape(bb, 4, 64), axis=1)
    feat = jnp.maximum(feat[:, 0:32], feat[:, 32:64])
    h = jnp.maximum(
        jnp.dot(feat, wh[...], preferred_element_type=f32) + bh[...], 0.0)
    o_ref[...] = jnp.dot(h, wo[...], preferred_element_type=f32) + bo[...]


def kernel(conv1_w, conv1_b, conv2_w, conv2_b, conv3_w, conv3_b,
           conv4_w, conv4_b, conv5_w, conv5_b, hidden_w, hidden_b,
           out_w, out_b, x):
    B = x.shape[0]
    nb = B // _BB
    xb = x.astype(jnp.bfloat16).reshape(nb, _BB, 32, 32, 32)

    t1 = _band(conv1_w, 32, 28, 32, 256)
    t2 = _band(conv2_w, 28, 24, 256)
    t3 = _band(conv3_w, 12, 9, 256, 384)
    t4 = _band(conv4_w, 9, 6, 384)
    t5 = _band(conv5_w, 3, 2, 256)
    b1 = _btile(conv1_b, 28, 256)
    b2 = _btile(conv2_b, 24)
    b3 = _btile(conv3_b, 9, 384)
    b4 = _btile(conv4_b, 6)
    b5 = _btile(conv5_b, 2)
    bh = hidden_b.reshape(1, -1).astype(jnp.float32)
    bo = out_b.reshape(1, -1).astype(jnp.float32)

    def cspec(arr):
        return pl.BlockSpec(arr.shape, lambda i: (0,) * arr.ndim)

    consts = [t1, b1, t2, b2, t3, b3, t4, b4, t5, b5,
              hidden_w, bh, out_w, bo]
    out = pl.pallas_call(
        _net_kernel,
        out_shape=jax.ShapeDtypeStruct((nb, _BB, 10), jnp.float32),
        grid=(nb,),
        in_specs=[pl.BlockSpec((None, _BB, 32, 32, 32),
                               lambda i: (i, 0, 0, 0, 0))]
                 + [cspec(a) for a in consts],
        out_specs=pl.BlockSpec((None, _BB, 10), lambda i: (i, 0, 0)),
        compiler_params=pltpu.CompilerParams(
            dimension_semantics=("parallel",),
            vmem_limit_bytes=64 * 1024 * 1024),
    )(xb, *consts)
    return out.reshape(B, 10)
